# baseline (speedup 1.0000x reference)
import functools

import jax
import jax.numpy as jnp
from jax import lax
from jax.experimental import pallas as pl
from jax.experimental.pallas import tpu as pltpu
from jax.experimental.pallas import tpu_sc as plsc

_N = 4096
_D = 128
_F = _N * _N // 128
_BM = 256

_NC = 2
_NS = 16
_NW = _NC * _NS
_PER_W = _N * _N // _NW
_CH = 16384
_NCH = _PER_W // _CH


def _norm_body(w_ref, out_ref):
    w = w_ref[...]
    n = jnp.sqrt(jnp.sum(w * w, axis=1, keepdims=True))
    out_ref[...] = w / jnp.maximum(n, 1e-8)


def _main_body(wi_ref, wall_ref, a_ref, attr_ref, *, bm):
    i = pl.program_id(0)
    a = jnp.dot(wi_ref[...], wall_ref[...].T, preferred_element_type=jnp.float32)
    row = jax.lax.broadcasted_iota(jnp.int32, (bm, _N), 0) + i * bm
    col = jax.lax.broadcasted_iota(jnp.int32, (bm, _N), 1)
    a = jnp.where(row == col, 0.0, jnp.maximum(a, 0.0))
    a_ref[...] = a
    attr_ref[...] = a.reshape(bm * 32, 128)


@functools.partial(
    pl.kernel,
    mesh=plsc.VectorSubcoreMesh(core_axis_name="c", subcore_axis_name="s"),
    out_type=jax.ShapeDtypeStruct((2, _N * _N), jnp.int32),
    scratch_types=[
        pltpu.VMEM((_CH,), jnp.int32),
        pltpu.VMEM((_CH,), jnp.int32),
    ],
)
def _idx_sc(out_hbm, src_v, dst_v):
    c = lax.axis_index("c")
    s = lax.axis_index("s")
    wid = s * _NC + c
    base = wid * _PER_W

    def chunk_body(ch, carry):
        off = base + ch * _CH

        def vec_body(j, carry2):
            k = off + j * 16 + lax.iota(jnp.int32, 16)
            src_v[pl.ds(j * 16, 16)] = k >> 12
            dst_v[pl.ds(j * 16, 16)] = k & (_N - 1)
            return carry2

        lax.fori_loop(0, _CH // 16, vec_body, 0)
        pltpu.sync_copy(src_v, out_hbm.at[0, pl.ds(off, _CH)])
        pltpu.sync_copy(dst_v, out_hbm.at[1, pl.ds(off, _CH)])
        return carry

    lax.fori_loop(0, _NCH, chunk_body, 0)


def kernel(W):
    Wn = pl.pallas_call(
        _norm_body,
        out_shape=jax.ShapeDtypeStruct((_N, _D), jnp.float32),
    )(W)

    a, attr = pl.pallas_call(
        functools.partial(_main_body, bm=_BM),
        grid=(_N // _BM,),
        in_specs=[
            pl.BlockSpec((_BM, _D), lambda i: (i, 0)),
            pl.BlockSpec((_N, _D), lambda i: (0, 0)),
        ],
        out_specs=[
            pl.BlockSpec((_BM, _N), lambda i: (i, 0)),
            pl.BlockSpec((_BM * 32, 128), lambda i: (i, 0)),
        ],
        out_shape=[
            jax.ShapeDtypeStruct((_N, _N), jnp.float32),
            jax.ShapeDtypeStruct((_F, 128), jnp.float32),
        ],
    )(Wn, Wn)

    edge_indices = _idx_sc()
    edge_attr = attr.reshape(_N * _N)
    return (edge_indices, edge_attr, a)
